# baseline (speedup 1.0000x reference)
import functools

import jax
import jax.numpy as jnp
from jax import lax
from jax.experimental import pallas as pl
from jax.experimental.pallas import tpu as pltpu
from jax.experimental.pallas import tpu_sc as plsc

E = 8
D = 768
F = 3072
N = 4096
T = 256
G = N // T + E
P = G * T
GP = 128
CH = 256
NCH = N // CH

NXT_OFF = 64

NC = 2
NS = 16
NW = NC * NS
TPW = N // NW


def _routing_body(feats_ref, cent_ref, dest_ref, te_ref, oh_ref, prefix_ref):
    feats = feats_ref[...]
    cent = cent_ref[...]
    scores = lax.dot_general(
        feats, cent, (((1,), (1,)), ((), ())),
        preferred_element_type=jnp.float32)

    iota_e = lax.broadcasted_iota(jnp.int32, (N, E), 1)
    mx = jnp.max(scores, axis=1, keepdims=True)
    te = jnp.min(jnp.where(scores == mx, iota_e, E), axis=1, keepdims=True)
    oh_ref[...] = (te == iota_e).astype(jnp.float32)

    r = lax.broadcasted_iota(jnp.int32, (CH, CH), 0)
    c = lax.broadcasted_iota(jnp.int32, (CH, CH), 1)
    ltri = (c < r).astype(jnp.float32)

    def step(i, carry):
        chunk = oh_ref[pl.ds(i * CH, CH), :]
        prefix_ref[pl.ds(i * CH, CH), :] = carry + lax.dot_general(
            ltri, chunk, (((1,), (0,)), ((), ())),
            preferred_element_type=jnp.float32)
        return carry + jnp.sum(chunk, axis=0, keepdims=True)

    counts_f = lax.fori_loop(0, NCH, step, jnp.zeros((1, E), jnp.float32))
    counts = counts_f.astype(jnp.int32)
    pc = ((counts + (T - 1)) // T) * T

    rr = lax.broadcasted_iota(jnp.int32, (E, E), 0)
    cc = lax.broadcasted_iota(jnp.int32, (E, E), 1)
    utri = (rr < cc).astype(jnp.float32)
    off_f = lax.dot_general(pc.astype(jnp.float32), utri,
                            (((1,), (0,)), ((), ())),
                            preferred_element_type=jnp.float32)
    off_i = off_f.astype(jnp.int32)

    def step2(i, _):
        pre = prefix_ref[pl.ds(i * CH, CH), :]
        ohc = oh_ref[pl.ds(i * CH, CH), :]
        d = jnp.sum((off_f + pre) * ohc, axis=1, keepdims=True)
        dest_ref[pl.ds(i * CH, CH), :] = d.astype(jnp.int32)
        return 0

    lax.fori_loop(0, NCH, step2, 0)

    tgrid = lax.broadcasted_iota(jnp.int32, (GP, 1), 0)
    tg2 = tgrid - NXT_OFF
    cc1 = lax.broadcasted_iota(jnp.int32, (1, E), 1)
    tex = jnp.zeros((GP, 1), jnp.int32)
    for e in range(E):
        s_e = lax.slice(off_i, (0, e), (1, e + 1)) // T
        n_e = lax.slice(pc, (0, e), (1, e + 1)) // T
        mask = (tgrid >= s_e) & (tgrid < s_e + n_e)
        tex = tex + e * mask.astype(jnp.int32)
        cand = jnp.where((cc1 > e) & (pc > 0), cc1, E)
        ne = jnp.min(cand, axis=1, keepdims=True)
        ne = jnp.where(ne == E, e, ne)
        mask2 = (tg2 >= s_e) & (tg2 < s_e + n_e)
        tex = tex + ne * mask2.astype(jnp.int32)
    un = jnp.sum(pc, axis=1, keepdims=True) // T
    tex = tex + (E - 1) * ((tgrid >= un) & (tgrid < G)).astype(jnp.int32)
    tex = tex + (tgrid == G).astype(jnp.int32) * un
    te_ref[...] = tex


def _routing(feats, cent):
    dest, te_tiles = pl.pallas_call(
        _routing_body,
        out_shape=[
            jax.ShapeDtypeStruct((N, 1), jnp.int32),
            jax.ShapeDtypeStruct((GP, 1), jnp.int32),
        ],
        scratch_shapes=[
            pltpu.VMEM((N, E), jnp.float32),
            pltpu.VMEM((N, E), jnp.float32),
        ],
    )(feats, cent)
    return dest.reshape(N), te_tiles.reshape(GP)


@functools.cache
def _sc_kernels():
    mesh = plsc.VectorSubcoreMesh(
        core_axis_name="c", subcore_axis_name="s",
        num_cores=NC, num_subcores=NS)
    scratch = [
        pltpu.VMEM((TPW,), jnp.int32),
        pltpu.VMEM((TPW, D), jnp.float32),
        pltpu.SemaphoreType.DMA,
    ]

    @functools.partial(
        pl.kernel,
        out_type=jax.ShapeDtypeStruct((P, D), jnp.float32),
        mesh=mesh, scratch_types=scratch)
    def scatter_k(feats_hbm, dest_hbm, out_hbm, idx_v, rows_v, sem):
        wid = lax.axis_index("s") * NC + lax.axis_index("c")
        base = wid * TPW
        c1 = pltpu.make_async_copy(dest_hbm.at[pl.ds(base, TPW)], idx_v, sem)
        c2 = pltpu.make_async_copy(feats_hbm.at[pl.ds(base, TPW)], rows_v, sem)
        c1.start()
        c2.start()
        c1.wait()
        c2.wait()
        pltpu.async_copy(rows_v, out_hbm.at[idx_v], sem).wait()

    @functools.partial(
        pl.kernel,
        out_type=jax.ShapeDtypeStruct((N, D), jnp.float32),
        mesh=mesh, scratch_types=scratch)
    def gather_k(sorted_hbm, dest_hbm, out_hbm, idx_v, rows_v, sem):
        wid = lax.axis_index("s") * NC + lax.axis_index("c")
        base = wid * TPW
        pltpu.sync_copy(dest_hbm.at[pl.ds(base, TPW)], idx_v)
        pltpu.async_copy(sorted_hbm.at[idx_v], rows_v, sem).wait()
        pltpu.sync_copy(rows_v, out_hbm.at[pl.ds(base, TPW)])

    return scatter_k, gather_k


def _ffn_body(s_ref, xb_ref, cent_ref, lng_ref, lnb_ref,
              w1_ref, b1_ref, w2_ref, b2_ref, out_ref,
              w1s_ref, w2s_ref, w1b_ref, w2b_ref, sem):
    t = pl.program_id(0)
    nu = s_ref[G]
    e_now = s_ref[t]
    e_prev = s_ref[jnp.maximum(t - 1, 0)]
    change = (t == 0) | (e_now != e_prev)

    def w_copies(e):
        return (pltpu.make_async_copy(w1_ref.at[e], w1s_ref, sem),
                pltpu.make_async_copy(w2_ref.at[e], w2s_ref, sem))

    @pl.when(t == 0)
    def _():
        c1, c2 = w_copies(e_now)
        c1.start()
        c2.start()

    @pl.when((t < nu) & change)
    def _():
        c1, c2 = w_copies(e_now)
        c1.wait()
        c2.wait()
        w1b_ref[...] = w1s_ref[...].astype(jnp.bfloat16)
        w2b_ref[...] = w2s_ref[...].astype(jnp.bfloat16)
        e_nxt = s_ref[NXT_OFF + t]

        @pl.when(e_nxt != e_now)
        def _():
            n1, n2 = w_copies(e_nxt)
            n1.start()
            n2.start()

    @pl.when(t < nu)
    def _run():
        xb = xb_ref[...]
        cb = cent_ref[0]
        score = jnp.sum(xb * cb, axis=1, keepdims=True)
        alpha = 1.0 / (1.0 + jnp.exp(-score))

        m = jnp.mean(xb, axis=1, keepdims=True)
        v = jnp.mean(xb * xb, axis=1, keepdims=True) - m * m
        h = (xb - m) / jnp.sqrt(v + 1e-5) * lng_ref[0] + lnb_ref[0]

        h1 = lax.dot_general(h.astype(jnp.bfloat16), w1b_ref[...],
                             (((1,), (1,)), ((), ())),
                             preferred_element_type=jnp.float32) + b1_ref[0]
        h1 = jnp.maximum(h1, 0.0)
        ffn = lax.dot_general(h1.astype(jnp.bfloat16), w2b_ref[...],
                              (((1,), (1,)), ((), ())),
                              preferred_element_type=jnp.float32) + b2_ref[0]
        out_ref[...] = xb + alpha * ffn


def _ffn(sorted_x, te_tiles, cent, ln_g, ln_b, w1, b1, w2, b2):
    grid_spec = pltpu.PrefetchScalarGridSpec(
        num_scalar_prefetch=1,
        grid=(G,),
        in_specs=[
            pl.BlockSpec((T, D), lambda t, s: (jnp.minimum(t, s[G] - 1), 0)),
            pl.BlockSpec((1, 1, D), lambda t, s: (s[t], 0, 0)),
            pl.BlockSpec((1, 1, D), lambda t, s: (s[t], 0, 0)),
            pl.BlockSpec((1, 1, D), lambda t, s: (s[t], 0, 0)),
            pl.BlockSpec(memory_space=pltpu.MemorySpace.HBM),
            pl.BlockSpec((1, 1, F), lambda t, s: (s[t], 0, 0)),
            pl.BlockSpec(memory_space=pltpu.MemorySpace.HBM),
            pl.BlockSpec((1, 1, D), lambda t, s: (s[t], 0, 0)),
        ],
        out_specs=pl.BlockSpec(
            (T, D), lambda t, s: (jnp.minimum(t, s[G] - 1), 0)),
        scratch_shapes=[
            pltpu.VMEM((F, D), jnp.float32),
            pltpu.VMEM((D, F), jnp.float32),
            pltpu.VMEM((F, D), jnp.bfloat16),
            pltpu.VMEM((D, F), jnp.bfloat16),
            pltpu.SemaphoreType.DMA,
        ],
    )
    return pl.pallas_call(
        _ffn_body,
        grid_spec=grid_spec,
        out_shape=jax.ShapeDtypeStruct((P, D), jnp.float32),
        compiler_params=pltpu.CompilerParams(
            vmem_limit_bytes=100 * 1024 * 1024),
    )(te_tiles, sorted_x,
      cent.reshape(E, 1, D), ln_g.reshape(E, 1, D), ln_b.reshape(E, 1, D),
      w1, b1.reshape(E, 1, F), w2, b2.reshape(E, 1, D))


def kernel(x, expert_centroids, ln_g, ln_b, w1, b1, w2, b2):
    feats = x.reshape(-1, D)
    scatter_k, gather_k = _sc_kernels()
    dest, te_tiles = _routing(feats, expert_centroids)
    sorted_x = scatter_k(feats, dest)
    sorted_out = _ffn(sorted_x, te_tiles, expert_centroids,
                      ln_g, ln_b, w1, b1, w2, b2)
    out = gather_k(sorted_out, dest)
    return out.reshape(x.shape)


# speedup vs baseline: 1.0068x; 1.0068x over previous
import functools

import jax
import jax.numpy as jnp
from jax import lax
from jax.experimental import pallas as pl
from jax.experimental.pallas import tpu as pltpu
from jax.experimental.pallas import tpu_sc as plsc

E = 8
D = 768
F = 3072
N = 4096
T = 256
G = N // T + E
P = G * T
GP = 128
CH = 512
NCH = N // CH

NXT_OFF = 64

NC = 2
NS = 16
NW = NC * NS
TPW = N // NW


def _routing_body(feats_ref, cent_ref, dest_ref, te_ref, oh_ref, prefix_ref):
    feats = feats_ref[...]
    cent = cent_ref[...]
    scores = lax.dot_general(
        feats, cent, (((1,), (1,)), ((), ())),
        preferred_element_type=jnp.float32)

    iota_e = lax.broadcasted_iota(jnp.int32, (N, E), 1)
    mx = jnp.max(scores, axis=1, keepdims=True)
    te = jnp.min(jnp.where(scores == mx, iota_e, E), axis=1, keepdims=True)
    oh_ref[...] = (te == iota_e).astype(jnp.float32)

    r = lax.broadcasted_iota(jnp.int32, (CH, CH), 0)
    c = lax.broadcasted_iota(jnp.int32, (CH, CH), 1)
    ltri = (c < r).astype(jnp.float32)

    def step(i, carry):
        chunk = oh_ref[pl.ds(i * CH, CH), :]
        prefix_ref[pl.ds(i * CH, CH), :] = carry + lax.dot_general(
            ltri, chunk, (((1,), (0,)), ((), ())),
            preferred_element_type=jnp.float32)
        return carry + jnp.sum(chunk, axis=0, keepdims=True)

    counts_f = lax.fori_loop(0, NCH, step, jnp.zeros((1, E), jnp.float32))
    counts = counts_f.astype(jnp.int32)
    pc = ((counts + (T - 1)) // T) * T

    rr = lax.broadcasted_iota(jnp.int32, (E, E), 0)
    cc = lax.broadcasted_iota(jnp.int32, (E, E), 1)
    utri = (rr < cc).astype(jnp.float32)
    off_f = lax.dot_general(pc.astype(jnp.float32), utri,
                            (((1,), (0,)), ((), ())),
                            preferred_element_type=jnp.float32)
    off_i = off_f.astype(jnp.int32)

    def step2(i, _):
        pre = prefix_ref[pl.ds(i * CH, CH), :]
        ohc = oh_ref[pl.ds(i * CH, CH), :]
        d = jnp.sum((off_f + pre) * ohc, axis=1, keepdims=True)
        dest_ref[pl.ds(i * CH, CH), :] = d.astype(jnp.int32)
        return 0

    lax.fori_loop(0, NCH, step2, 0)

    tgrid = lax.broadcasted_iota(jnp.int32, (GP, 1), 0)
    tg2 = tgrid - NXT_OFF
    cc1 = lax.broadcasted_iota(jnp.int32, (1, E), 1)
    tex = jnp.zeros((GP, 1), jnp.int32)
    for e in range(E):
        s_e = lax.slice(off_i, (0, e), (1, e + 1)) // T
        n_e = lax.slice(pc, (0, e), (1, e + 1)) // T
        mask = (tgrid >= s_e) & (tgrid < s_e + n_e)
        tex = tex + e * mask.astype(jnp.int32)
        cand = jnp.where((cc1 > e) & (pc > 0), cc1, E)
        ne = jnp.min(cand, axis=1, keepdims=True)
        ne = jnp.where(ne == E, e, ne)
        mask2 = (tg2 >= s_e) & (tg2 < s_e + n_e)
        tex = tex + ne * mask2.astype(jnp.int32)
    un = jnp.sum(pc, axis=1, keepdims=True) // T
    tex = tex + (E - 1) * ((tgrid >= un) & (tgrid < G)).astype(jnp.int32)
    tex = tex + (tgrid == G).astype(jnp.int32) * un
    te_ref[...] = tex


def _routing(feats, cent):
    dest, te_tiles = pl.pallas_call(
        _routing_body,
        out_shape=[
            jax.ShapeDtypeStruct((N, 1), jnp.int32),
            jax.ShapeDtypeStruct((GP, 1), jnp.int32),
        ],
        scratch_shapes=[
            pltpu.VMEM((N, E), jnp.float32),
            pltpu.VMEM((N, E), jnp.float32),
        ],
    )(feats, cent)
    return dest.reshape(N), te_tiles.reshape(GP)


@functools.cache
def _sc_kernels():
    mesh = plsc.VectorSubcoreMesh(
        core_axis_name="c", subcore_axis_name="s",
        num_cores=NC, num_subcores=NS)
    scratch = [
        pltpu.VMEM((TPW,), jnp.int32),
        pltpu.VMEM((TPW, D), jnp.float32),
        pltpu.SemaphoreType.DMA,
    ]

    @functools.partial(
        pl.kernel,
        out_type=jax.ShapeDtypeStruct((P, D), jnp.float32),
        mesh=mesh, scratch_types=scratch)
    def scatter_k(feats_hbm, dest_hbm, out_hbm, idx_v, rows_v, sem):
        wid = lax.axis_index("s") * NC + lax.axis_index("c")
        base = wid * TPW
        pltpu.sync_copy(dest_hbm.at[pl.ds(base, TPW)], idx_v)
        pltpu.sync_copy(feats_hbm.at[pl.ds(base, TPW)], rows_v)
        pltpu.async_copy(rows_v, out_hbm.at[idx_v], sem).wait()

    @functools.partial(
        pl.kernel,
        out_type=jax.ShapeDtypeStruct((N, D), jnp.float32),
        mesh=mesh, scratch_types=scratch)
    def gather_k(sorted_hbm, dest_hbm, out_hbm, idx_v, rows_v, sem):
        wid = lax.axis_index("s") * NC + lax.axis_index("c")
        base = wid * TPW
        pltpu.sync_copy(dest_hbm.at[pl.ds(base, TPW)], idx_v)
        pltpu.async_copy(sorted_hbm.at[idx_v], rows_v, sem).wait()
        pltpu.sync_copy(rows_v, out_hbm.at[pl.ds(base, TPW)])

    return scatter_k, gather_k


def _ffn_body(s_ref, xb_ref, cent_ref, lng_ref, lnb_ref,
              w1_ref, b1_ref, w2_ref, b2_ref, out_ref,
              w1s_ref, w2s_ref, w1b_ref, w2b_ref, sem):
    t = pl.program_id(0)
    nu = s_ref[G]
    e_now = s_ref[t]
    e_prev = s_ref[jnp.maximum(t - 1, 0)]
    change = (t == 0) | (e_now != e_prev)

    def w_copies(e):
        return (pltpu.make_async_copy(w1_ref.at[e], w1s_ref, sem),
                pltpu.make_async_copy(w2_ref.at[e], w2s_ref, sem))

    @pl.when(t == 0)
    def _():
        c1, c2 = w_copies(e_now)
        c1.start()
        c2.start()

    @pl.when((t < nu) & change)
    def _():
        c1, c2 = w_copies(e_now)
        c1.wait()
        c2.wait()
        w1b_ref[...] = w1s_ref[...].astype(jnp.bfloat16)
        w2b_ref[...] = w2s_ref[...].astype(jnp.bfloat16)
        e_nxt = s_ref[NXT_OFF + t]

        @pl.when(e_nxt != e_now)
        def _():
            n1, n2 = w_copies(e_nxt)
            n1.start()
            n2.start()

    @pl.when(t < nu)
    def _run():
        xb = xb_ref[...]
        cb = cent_ref[0]
        score = jnp.sum(xb * cb, axis=1, keepdims=True)
        alpha = 1.0 / (1.0 + jnp.exp(-score))

        m = jnp.mean(xb, axis=1, keepdims=True)
        v = jnp.mean(xb * xb, axis=1, keepdims=True) - m * m
        h = (xb - m) / jnp.sqrt(v + 1e-5) * lng_ref[0] + lnb_ref[0]

        h1 = lax.dot_general(h.astype(jnp.bfloat16), w1b_ref[...],
                             (((1,), (1,)), ((), ())),
                             preferred_element_type=jnp.float32) + b1_ref[0]
        h1 = jnp.maximum(h1, 0.0)
        ffn = lax.dot_general(h1.astype(jnp.bfloat16), w2b_ref[...],
                              (((1,), (1,)), ((), ())),
                              preferred_element_type=jnp.float32) + b2_ref[0]
        out_ref[...] = xb + alpha * ffn


def _ffn(sorted_x, te_tiles, cent, ln_g, ln_b, w1, b1, w2, b2):
    grid_spec = pltpu.PrefetchScalarGridSpec(
        num_scalar_prefetch=1,
        grid=(G,),
        in_specs=[
            pl.BlockSpec((T, D), lambda t, s: (jnp.minimum(t, s[G] - 1), 0)),
            pl.BlockSpec((1, 1, D), lambda t, s: (s[t], 0, 0)),
            pl.BlockSpec((1, 1, D), lambda t, s: (s[t], 0, 0)),
            pl.BlockSpec((1, 1, D), lambda t, s: (s[t], 0, 0)),
            pl.BlockSpec(memory_space=pltpu.MemorySpace.HBM),
            pl.BlockSpec((1, 1, F), lambda t, s: (s[t], 0, 0)),
            pl.BlockSpec(memory_space=pltpu.MemorySpace.HBM),
            pl.BlockSpec((1, 1, D), lambda t, s: (s[t], 0, 0)),
        ],
        out_specs=pl.BlockSpec(
            (T, D), lambda t, s: (jnp.minimum(t, s[G] - 1), 0)),
        scratch_shapes=[
            pltpu.VMEM((F, D), jnp.float32),
            pltpu.VMEM((D, F), jnp.float32),
            pltpu.VMEM((F, D), jnp.bfloat16),
            pltpu.VMEM((D, F), jnp.bfloat16),
            pltpu.SemaphoreType.DMA,
        ],
    )
    return pl.pallas_call(
        _ffn_body,
        grid_spec=grid_spec,
        out_shape=jax.ShapeDtypeStruct((P, D), jnp.float32),
        compiler_params=pltpu.CompilerParams(
            vmem_limit_bytes=100 * 1024 * 1024),
    )(te_tiles, sorted_x,
      cent.reshape(E, 1, D), ln_g.reshape(E, 1, D), ln_b.reshape(E, 1, D),
      w1, b1.reshape(E, 1, F), w2, b2.reshape(E, 1, D))


def kernel(x, expert_centroids, ln_g, ln_b, w1, b1, w2, b2):
    feats = x.reshape(-1, D)
    scatter_k, gather_k = _sc_kernels()
    dest, te_tiles = _routing(feats, expert_centroids)
    sorted_x = scatter_k(feats, dest)
    sorted_out = _ffn(sorted_x, te_tiles, expert_centroids,
                      ln_g, ln_b, w1, b1, w2, b2)
    out = gather_k(sorted_out, dest)
    return out.reshape(x.shape)


# speedup vs baseline: 1.0237x; 1.0167x over previous
import functools

import jax
import jax.numpy as jnp
from jax import lax
from jax.experimental import pallas as pl
from jax.experimental.pallas import tpu as pltpu
from jax.experimental.pallas import tpu_sc as plsc

E = 8
D = 768
F = 3072
N = 4096
T = 256
G = N // T + E
P = G * T
GP = 128
CH = 512
NCH = N // CH

NXT_OFF = 64

NC = 2
NS = 16
NW = NC * NS
TPW = N // NW


def _routing_body(feats_ref, cent_ref, dest_ref, te_ref, oh_ref, prefix_ref):
    feats = feats_ref[...]
    cent = cent_ref[...]
    scores = lax.dot_general(
        feats, cent, (((1,), (1,)), ((), ())),
        preferred_element_type=jnp.float32)

    iota_e = lax.broadcasted_iota(jnp.int32, (N, E), 1)
    mx = jnp.max(scores, axis=1, keepdims=True)
    te = jnp.min(jnp.where(scores == mx, iota_e, E), axis=1, keepdims=True)
    oh_ref[...] = (te == iota_e).astype(jnp.float32)

    r = lax.broadcasted_iota(jnp.int32, (CH, CH), 0)
    c = lax.broadcasted_iota(jnp.int32, (CH, CH), 1)
    ltri = (c < r).astype(jnp.float32)

    def step(i, carry):
        chunk = oh_ref[pl.ds(i * CH, CH), :]
        prefix_ref[pl.ds(i * CH, CH), :] = carry + lax.dot_general(
            ltri, chunk, (((1,), (0,)), ((), ())),
            preferred_element_type=jnp.float32)
        return carry + jnp.sum(chunk, axis=0, keepdims=True)

    counts_f = lax.fori_loop(0, NCH, step, jnp.zeros((1, E), jnp.float32))
    counts = counts_f.astype(jnp.int32)
    pc = ((counts + (T - 1)) // T) * T

    rr = lax.broadcasted_iota(jnp.int32, (E, E), 0)
    cc = lax.broadcasted_iota(jnp.int32, (E, E), 1)
    utri = (rr < cc).astype(jnp.float32)
    off_f = lax.dot_general(pc.astype(jnp.float32), utri,
                            (((1,), (0,)), ((), ())),
                            preferred_element_type=jnp.float32)
    off_i = off_f.astype(jnp.int32)

    def step2(i, _):
        pre = prefix_ref[pl.ds(i * CH, CH), :]
        ohc = oh_ref[pl.ds(i * CH, CH), :]
        d = jnp.sum((off_f + pre) * ohc, axis=1, keepdims=True)
        dest_ref[pl.ds(i * (CH // 128), CH // 128), :] = (
            d.astype(jnp.int32).reshape(CH // 128, 128))
        return 0

    lax.fori_loop(0, NCH, step2, 0)

    tgrid = lax.broadcasted_iota(jnp.int32, (GP, 1), 0)
    tg2 = tgrid - NXT_OFF
    cc1 = lax.broadcasted_iota(jnp.int32, (1, E), 1)
    tex = jnp.zeros((GP, 1), jnp.int32)
    for e in range(E):
        s_e = lax.slice(off_i, (0, e), (1, e + 1)) // T
        n_e = lax.slice(pc, (0, e), (1, e + 1)) // T
        mask = (tgrid >= s_e) & (tgrid < s_e + n_e)
        tex = tex + e * mask.astype(jnp.int32)
        cand = jnp.where((cc1 > e) & (pc > 0), cc1, E)
        ne = jnp.min(cand, axis=1, keepdims=True)
        ne = jnp.where(ne == E, e, ne)
        mask2 = (tg2 >= s_e) & (tg2 < s_e + n_e)
        tex = tex + ne * mask2.astype(jnp.int32)
    un = jnp.sum(pc, axis=1, keepdims=True) // T
    tex = tex + (E - 1) * ((tgrid >= un) & (tgrid < G)).astype(jnp.int32)
    tex = tex + (tgrid == G).astype(jnp.int32) * un
    te_ref[...] = tex


def _routing(feats, cent):
    dest, te_tiles = pl.pallas_call(
        _routing_body,
        out_shape=[
            jax.ShapeDtypeStruct((N // 128, 128), jnp.int32),
            jax.ShapeDtypeStruct((GP, 1), jnp.int32),
        ],
        scratch_shapes=[
            pltpu.VMEM((N, E), jnp.float32),
            pltpu.VMEM((N, E), jnp.float32),
        ],
    )(feats, cent)
    return dest.reshape(N), te_tiles.reshape(GP)


@functools.cache
def _sc_kernels():
    mesh = plsc.VectorSubcoreMesh(
        core_axis_name="c", subcore_axis_name="s",
        num_cores=NC, num_subcores=NS)
    scratch = [
        pltpu.VMEM((TPW,), jnp.int32),
        pltpu.VMEM((TPW, D), jnp.float32),
        pltpu.SemaphoreType.DMA,
    ]

    @functools.partial(
        pl.kernel,
        out_type=jax.ShapeDtypeStruct((P, D), jnp.float32),
        mesh=mesh, scratch_types=scratch)
    def scatter_k(feats_hbm, dest_hbm, out_hbm, idx_v, rows_v, sem):
        wid = lax.axis_index("s") * NC + lax.axis_index("c")
        base = wid * TPW
        pltpu.sync_copy(dest_hbm.at[pl.ds(base, TPW)], idx_v)
        pltpu.sync_copy(feats_hbm.at[pl.ds(base, TPW)], rows_v)
        pltpu.async_copy(rows_v, out_hbm.at[idx_v], sem).wait()

    @functools.partial(
        pl.kernel,
        out_type=jax.ShapeDtypeStruct((N, D), jnp.float32),
        mesh=mesh, scratch_types=scratch)
    def gather_k(sorted_hbm, dest_hbm, out_hbm, idx_v, rows_v, sem):
        wid = lax.axis_index("s") * NC + lax.axis_index("c")
        base = wid * TPW
        pltpu.sync_copy(dest_hbm.at[pl.ds(base, TPW)], idx_v)
        pltpu.async_copy(sorted_hbm.at[idx_v], rows_v, sem).wait()
        pltpu.sync_copy(rows_v, out_hbm.at[pl.ds(base, TPW)])

    return scatter_k, gather_k


def _ffn_body(s_ref, xb_ref, cent_ref, lng_ref, lnb_ref,
              w1_ref, b1_ref, w2_ref, b2_ref, out_ref,
              w1s_ref, w2s_ref, w1b_ref, w2b_ref, sem):
    t = pl.program_id(0)
    nu = s_ref[G]
    e_now = s_ref[t]
    e_prev = s_ref[jnp.maximum(t - 1, 0)]
    change = (t == 0) | (e_now != e_prev)

    def w_copies(e):
        return (pltpu.make_async_copy(w1_ref.at[e], w1s_ref, sem),
                pltpu.make_async_copy(w2_ref.at[e], w2s_ref, sem))

    @pl.when(t == 0)
    def _():
        c1, c2 = w_copies(e_now)
        c1.start()
        c2.start()

    @pl.when((t < nu) & change)
    def _():
        c1, c2 = w_copies(e_now)
        c1.wait()
        c2.wait()
        w1b_ref[...] = w1s_ref[...].astype(jnp.bfloat16)
        w2b_ref[...] = w2s_ref[...].astype(jnp.bfloat16)
        e_nxt = s_ref[NXT_OFF + t]

        @pl.when(e_nxt != e_now)
        def _():
            n1, n2 = w_copies(e_nxt)
            n1.start()
            n2.start()

    @pl.when(t < nu)
    def _run():
        xb = xb_ref[...]
        cb = cent_ref[0]
        score = jnp.sum(xb * cb, axis=1, keepdims=True)
        alpha = 1.0 / (1.0 + jnp.exp(-score))

        m = jnp.mean(xb, axis=1, keepdims=True)
        v = jnp.mean(xb * xb, axis=1, keepdims=True) - m * m
        h = (xb - m) / jnp.sqrt(v + 1e-5) * lng_ref[0] + lnb_ref[0]

        h1 = lax.dot_general(h.astype(jnp.bfloat16), w1b_ref[...],
                             (((1,), (1,)), ((), ())),
                             preferred_element_type=jnp.float32) + b1_ref[0]
        h1 = jnp.maximum(h1, 0.0)
        ffn = lax.dot_general(h1.astype(jnp.bfloat16), w2b_ref[...],
                              (((1,), (1,)), ((), ())),
                              preferred_element_type=jnp.float32) + b2_ref[0]
        out_ref[...] = xb + alpha * ffn


def _ffn(sorted_x, te_tiles, cent, ln_g, ln_b, w1, b1, w2, b2):
    grid_spec = pltpu.PrefetchScalarGridSpec(
        num_scalar_prefetch=1,
        grid=(G,),
        in_specs=[
            pl.BlockSpec((T, D), lambda t, s: (jnp.minimum(t, s[G] - 1), 0)),
            pl.BlockSpec((1, 1, D), lambda t, s: (s[t], 0, 0)),
            pl.BlockSpec((1, 1, D), lambda t, s: (s[t], 0, 0)),
            pl.BlockSpec((1, 1, D), lambda t, s: (s[t], 0, 0)),
            pl.BlockSpec(memory_space=pltpu.MemorySpace.HBM),
            pl.BlockSpec((1, 1, F), lambda t, s: (s[t], 0, 0)),
            pl.BlockSpec(memory_space=pltpu.MemorySpace.HBM),
            pl.BlockSpec((1, 1, D), lambda t, s: (s[t], 0, 0)),
        ],
        out_specs=pl.BlockSpec(
            (T, D), lambda t, s: (jnp.minimum(t, s[G] - 1), 0)),
        scratch_shapes=[
            pltpu.VMEM((F, D), jnp.float32),
            pltpu.VMEM((D, F), jnp.float32),
            pltpu.VMEM((F, D), jnp.bfloat16),
            pltpu.VMEM((D, F), jnp.bfloat16),
            pltpu.SemaphoreType.DMA,
        ],
    )
    return pl.pallas_call(
        _ffn_body,
        grid_spec=grid_spec,
        out_shape=jax.ShapeDtypeStruct((P, D), jnp.float32),
        compiler_params=pltpu.CompilerParams(
            vmem_limit_bytes=100 * 1024 * 1024),
    )(te_tiles, sorted_x,
      cent.reshape(E, 1, D), ln_g.reshape(E, 1, D), ln_b.reshape(E, 1, D),
      w1, b1.reshape(E, 1, F), w2, b2.reshape(E, 1, D))


def kernel(x, expert_centroids, ln_g, ln_b, w1, b1, w2, b2):
    feats = x.reshape(-1, D)
    scatter_k, gather_k = _sc_kernels()
    dest, te_tiles = _routing(feats, expert_centroids)
    sorted_x = scatter_k(feats, dest)
    sorted_out = _ffn(sorted_x, te_tiles, expert_centroids,
                      ln_g, ln_b, w1, b1, w2, b2)
    out = gather_k(sorted_out, dest)
    return out.reshape(x.shape)
